# Initial kernel scaffold; baseline (speedup 1.0000x reference)
import jax
import jax.numpy as jnp
from jax.experimental import pallas as pl


def kernel(actual_residual, predicted_residual, o_ce, m_cu, beta_ce, beta_cu):
    raise NotImplementedError("write your pallas kernel here")


# speedup vs baseline: 1.5523x; 1.5523x over previous
import functools

import jax
import jax.numpy as jnp
from jax.experimental import pallas as pl
from jax.experimental.pallas import tpu as pltpu

_CAPACITY = 0.5
_BLK_T = 256


def _moments_kernel(a_ref, p_ref, dst_ref, dch_ref, *, inv_d):
    a = a_ref[...]
    p = p_ref[...]
    d = a - p
    dst_ref[...] = jnp.sum(a * a, axis=-1) * inv_d
    dch_ref[...] = jnp.sum(d * d, axis=-1) * inv_d


def _gate_kernel(scal_ref, dst_ref, dch_ref, g_ref, bin_ref, *, k):
    dst = dst_ref[...]
    dch = dch_ref[...]
    b, t = dst.shape
    log_oce = scal_ref[0]
    m_cu = scal_ref[1]
    bce_pos = scal_ref[2]
    bcu_pos = scal_ref[3]

    ce = dst - (dch - log_oce)
    ma = jnp.mean(dst)
    cu = dst - m_cu * ma
    s_ce = jax.nn.sigmoid(bce_pos * ce)
    s_cu = jax.nn.sigmoid(bcu_pos * cu)
    g = s_ce + s_cu - s_ce * s_cu
    g_ref[...] = g

    bits = jax.lax.bitcast_convert_type(g, jnp.int32)

    def vbody(_, carry):
        lo, hi = carry
        mid = lo + ((hi - lo) >> 1)
        cnt = jnp.sum((bits >= mid).astype(jnp.int32), axis=1, keepdims=True)
        feas = cnt >= k
        return jnp.where(feas, mid, lo), jnp.where(feas, hi, mid)

    lo0 = jnp.zeros((b, 1), jnp.int32)
    hi0 = jnp.full((b, 1), jnp.int32(0x40000001))
    tbits, _ = jax.lax.fori_loop(0, 31, vbody, (lo0, hi0))

    gt = bits > tbits
    eq = bits == tbits
    need = k - jnp.sum(gt.astype(jnp.int32), axis=1, keepdims=True)
    iota = jax.lax.broadcasted_iota(jnp.int32, (b, t), 1)
    eqi = eq.astype(jnp.int32)

    def ibody(_, carry):
        lo, hi = carry
        mid = lo + ((hi - lo) >> 1)
        cnt = jnp.sum(eqi * (iota < mid).astype(jnp.int32), axis=1,
                      keepdims=True)
        geq = cnt >= need
        return jnp.where(geq, lo, mid), jnp.where(geq, mid, hi)

    lo0 = jnp.zeros((b, 1), jnp.int32)
    hi0 = jnp.full((b, 1), jnp.int32(t))
    _, jstar = jax.lax.fori_loop(0, 14, ibody, (lo0, hi0))

    bin_ref[...] = (gt | (eq & (iota < jstar))).astype(jnp.float32)


def kernel(actual_residual, predicted_residual, o_ce, m_cu, beta_ce, beta_cu):
    bv, tv, dv = actual_residual.shape
    k = max(1, int(tv * _CAPACITY))

    dst, dch = pl.pallas_call(
        functools.partial(_moments_kernel, inv_d=1.0 / dv),
        grid=(tv // _BLK_T,),
        in_specs=[
            pl.BlockSpec((bv, _BLK_T, dv), lambda i: (0, i, 0)),
            pl.BlockSpec((bv, _BLK_T, dv), lambda i: (0, i, 0)),
        ],
        out_specs=[
            pl.BlockSpec((bv, _BLK_T), lambda i: (0, i)),
            pl.BlockSpec((bv, _BLK_T), lambda i: (0, i)),
        ],
        out_shape=[
            jax.ShapeDtypeStruct((bv, tv), jnp.float32),
            jax.ShapeDtypeStruct((bv, tv), jnp.float32),
        ],
        compiler_params=pltpu.CompilerParams(
            dimension_semantics=("parallel",),
        ),
    )(actual_residual, predicted_residual)

    scal = jnp.stack([
        jnp.log(o_ce + 1e-10),
        m_cu,
        jax.nn.softplus(beta_ce),
        jax.nn.softplus(beta_cu),
    ]).astype(jnp.float32)

    g, binary = pl.pallas_call(
        functools.partial(_gate_kernel, k=k),
        in_specs=[
            pl.BlockSpec(memory_space=pltpu.SMEM),
            pl.BlockSpec((bv, tv), lambda: (0, 0)),
            pl.BlockSpec((bv, tv), lambda: (0, 0)),
        ],
        out_specs=[
            pl.BlockSpec((bv, tv), lambda: (0, 0)),
            pl.BlockSpec((bv, tv), lambda: (0, 0)),
        ],
        out_shape=[
            jax.ShapeDtypeStruct((bv, tv), jnp.float32),
            jax.ShapeDtypeStruct((bv, tv), jnp.float32),
        ],
    )(scal, dst, dch)

    return (g, binary)
